# Initial kernel scaffold; baseline (speedup 1.0000x reference)
import jax
import jax.numpy as jnp
from jax.experimental import pallas as pl


def kernel(words, tags, word_table, tag_table):
    raise NotImplementedError("write your pallas kernel here")


# speedup vs baseline: 3.3649x; 3.3649x over previous
import functools

import jax
import jax.numpy as jnp
from jax import lax
from jax.experimental import pallas as pl
from jax.experimental.pallas import tpu as pltpu
from jax.experimental.pallas import tpu_sc as plsc

WORD_DIM = 64
TAG_DIM = 32
OUT_DIM = WORD_DIM + TAG_DIM

CHUNK = 128
SB = 1024
NCHUNK = SB // CHUNK


def _build_kernel(N, num_cores, num_subcores):
  NW = num_cores * num_subcores
  per_w = N // NW
  n_sb = per_w // SB
  idx_rows_per_w = per_w // CHUNK

  mesh = plsc.VectorSubcoreMesh(core_axis_name="c", subcore_axis_name="s")

  @functools.partial(
      pl.kernel,
      mesh=mesh,
      out_type=jax.ShapeDtypeStruct((N, OUT_DIM), jnp.float32),
      compiler_params=pltpu.CompilerParams(use_tc_tiling_on_sc=False),
      scratch_types=[
          pltpu.VMEM((NCHUNK, CHUNK), jnp.int32),
          pltpu.VMEM((NCHUNK, CHUNK), jnp.int32),
          pltpu.VMEM((SB, WORD_DIM), jnp.float32),
          pltpu.VMEM((SB, TAG_DIM), jnp.float32),
          pltpu.SemaphoreType.DMA,
          pltpu.SemaphoreType.DMA,
      ],
  )
  def k(w_hbm, t_hbm, wt_hbm, tt_hbm, out_hbm,
        widx, tidx, wrows, trows, wsem, tsem):
    c = lax.axis_index("c")
    s = lax.axis_index("s")
    wid = s * num_cores + c
    idx_base = wid * idx_rows_per_w
    row_base = wid * per_w

    def body(sb, carry):
      pltpu.sync_copy(w_hbm.at[pl.ds(idx_base + sb * NCHUNK, NCHUNK)], widx)
      pltpu.sync_copy(t_hbm.at[pl.ds(idx_base + sb * NCHUNK, NCHUNK)], tidx)
      copies = []
      for j in range(NCHUNK):
        copies.append(pltpu.async_copy(
            wt_hbm.at[widx.at[j]],
            wrows.at[pl.ds(j * CHUNK, CHUNK)], wsem))
        copies.append(pltpu.async_copy(
            tt_hbm.at[tidx.at[j]],
            trows.at[pl.ds(j * CHUNK, CHUNK)], tsem))
      for cp in copies:
        cp.wait()
      off = row_base + sb * SB
      pltpu.sync_copy(wrows, out_hbm.at[pl.ds(off, SB), pl.ds(0, WORD_DIM)])
      pltpu.sync_copy(trows, out_hbm.at[pl.ds(off, SB), pl.ds(WORD_DIM, TAG_DIM)])
      return carry

    lax.fori_loop(0, n_sb, body, 0)

  return k


def kernel(words, tags, word_table, tag_table):
  B, L = words.shape
  N = B * L
  info = plsc.get_sparse_core_info()
  k = _build_kernel(N, info.num_cores, info.num_subcores)
  w2 = words.reshape(N // CHUNK, CHUNK)
  t2 = tags.reshape(N // CHUNK, CHUNK)
  out = k(w2, t2, word_table, tag_table)
  return out.reshape(B, L, OUT_DIM)


# speedup vs baseline: 8.0076x; 2.3797x over previous
import functools

import jax
import jax.numpy as jnp
from jax import lax
from jax.experimental import pallas as pl
from jax.experimental.pallas import tpu as pltpu
from jax.experimental.pallas import tpu_sc as plsc

WORD_DIM = 64
TAG_DIM = 32
OUT_DIM = WORD_DIM + TAG_DIM

CHUNK = 128
SB = 512
NCHUNK = SB // CHUNK
NBUF = 2


def _build_kernel(N, num_cores, num_subcores):
  NW = num_cores * num_subcores
  per_w = N // NW
  n_sb = per_w // SB
  n_body = n_sb // NBUF
  idx_rows_per_w = per_w // CHUNK

  mesh = plsc.VectorSubcoreMesh(core_axis_name="c", subcore_axis_name="s")

  @functools.partial(
      pl.kernel,
      mesh=mesh,
      out_type=jax.ShapeDtypeStruct((N, OUT_DIM), jnp.float32),
      compiler_params=pltpu.CompilerParams(use_tc_tiling_on_sc=False),
      scratch_types=[
          pltpu.VMEM((NBUF * NCHUNK, CHUNK), jnp.int32),
          pltpu.VMEM((NBUF * NCHUNK, CHUNK), jnp.int32),
          pltpu.VMEM((NBUF * SB, WORD_DIM), jnp.float32),
          pltpu.VMEM((NBUF * SB, TAG_DIM), jnp.float32),
          pltpu.SemaphoreType.DMA,
          pltpu.SemaphoreType.DMA,
          pltpu.SemaphoreType.DMA,
          pltpu.SemaphoreType.DMA,
      ],
  )
  def k(w_hbm, t_hbm, wt_hbm, tt_hbm, out_hbm,
        widx, tidx, wrows, trows, g0, g1, o0, o1):
    c = lax.axis_index("c")
    s = lax.axis_index("s")
    wid = s * num_cores + c
    idx_base = wid * idx_rows_per_w
    row_base = wid * per_w
    gsem = (g0, g1)
    osem = (o0, o1)

    def stage(sb, buf):
      pltpu.sync_copy(
          w_hbm.at[pl.ds(idx_base + sb * NCHUNK, NCHUNK)],
          widx.at[pl.ds(buf * NCHUNK, NCHUNK)])
      pltpu.sync_copy(
          t_hbm.at[pl.ds(idx_base + sb * NCHUNK, NCHUNK)],
          tidx.at[pl.ds(buf * NCHUNK, NCHUNK)])

    def fire(buf):
      copies = []
      for j in range(NCHUNK):
        copies.append(pltpu.async_copy(
            wt_hbm.at[widx.at[buf * NCHUNK + j]],
            wrows.at[pl.ds(buf * SB + j * CHUNK, CHUNK)], gsem[buf]))
        copies.append(pltpu.async_copy(
            tt_hbm.at[tidx.at[buf * NCHUNK + j]],
            trows.at[pl.ds(buf * SB + j * CHUNK, CHUNK)], gsem[buf]))
      return copies

    def write(sb, buf):
      off = row_base + sb * SB
      return [
          pltpu.async_copy(
              wrows.at[pl.ds(buf * SB, SB)],
              out_hbm.at[pl.ds(off, SB), pl.ds(0, WORD_DIM)], osem[buf]),
          pltpu.async_copy(
              trows.at[pl.ds(buf * SB, SB)],
              out_hbm.at[pl.ds(off, SB), pl.ds(WORD_DIM, TAG_DIM)], osem[buf]),
      ]

    def body(i, carry):
      sb0 = i * NBUF
      sb1 = sb0 + 1
      stage(sb0, 0)
      c0 = fire(0)
      stage(sb1, 1)
      c1 = fire(1)
      for cp in c0:
        cp.wait()
      w0 = write(sb0, 0)
      for cp in c1:
        cp.wait()
      w1 = write(sb1, 1)
      for cp in w0 + w1:
        cp.wait()
      return carry

    lax.fori_loop(0, n_body, body, 0)

  return k


def kernel(words, tags, word_table, tag_table):
  B, L = words.shape
  N = B * L
  info = plsc.get_sparse_core_info()
  k = _build_kernel(N, info.num_cores, info.num_subcores)
  w2 = words.reshape(N // CHUNK, CHUNK)
  t2 = tags.reshape(N // CHUNK, CHUNK)
  out = k(w2, t2, word_table, tag_table)
  return out.reshape(B, L, OUT_DIM)
